# baseline (speedup 1.0000x reference)
import functools

import jax
import jax.numpy as jnp
from jax import lax
from jax.experimental import pallas as pl
from jax.experimental.pallas import tpu as pltpu
from jax.experimental.pallas import tpu_sc as plsc

_NC = 2
_NS = 16
_L = 16
_NW = _NC * _NS
_B = 16384
_K = 32
_BPW = _B // _NW
_NG = _BPW // _L
_IC = 128


def _lfm_body(u_hbm, i_hbm, r_hbm, a_hbm, al_hbm, ag_hbm,
              bu_hbm, bi_hbm, gu_hbm, gi_hbm, out_hbm,
              idxu_v, idxi_v, r_v, a_v, al_v, ag_v,
              bu_v, bi_v, gu_v, gi_v, tbuf, accbuf, sem):
    wid = lax.axis_index("s") * _NC + lax.axis_index("c")
    base = wid * _BPW

    pltpu.sync_copy(u_hbm.at[pl.ds(base, _BPW)], idxu_v)
    pltpu.sync_copy(i_hbm.at[pl.ds(base, _BPW)], idxi_v)
    pltpu.sync_copy(r_hbm.at[pl.ds(base, _BPW)], r_v)
    pltpu.sync_copy(a_hbm.at[pl.ds(base, _BPW)], a_v)
    pltpu.sync_copy(al_hbm, al_v)
    pltpu.sync_copy(ag_hbm, ag_v)

    copies = []
    for j in range(_BPW // _IC):
        s = pl.ds(j * _IC, _IC)
        copies.append(pltpu.async_copy(gu_hbm.at[idxu_v.at[s]], gu_v.at[s], sem))
        copies.append(pltpu.async_copy(gi_hbm.at[idxi_v.at[s]], gi_v.at[s], sem))
        copies.append(pltpu.async_copy(bu_hbm.at[idxu_v.at[s]], bu_v.at[s], sem))
        copies.append(pltpu.async_copy(bi_hbm.at[idxi_v.at[s]], bi_v.at[s], sem))
    for c in copies:
        c.wait()

    alpha = al_v[...]
    agec = ag_v[...]
    scat_base = lax.iota(jnp.int32, _L) * _L

    def group(g, acc):
        row0 = g * _L
        for s in range(_L):
            row = row0 + s
            w = (gu_v[row, pl.ds(0, _L)] * gi_v[row, pl.ds(0, _L)]
                 + gu_v[row, pl.ds(_L, _L)] * gi_v[row, pl.ds(_L, _L)])
            plsc.store_scatter(tbuf, [scat_base + s], w)
        dot = tbuf[pl.ds(0, _L)]
        for d in range(1, _L):
            dot = dot + tbuf[pl.ds(d * _L, _L)]
        chunk = pl.ds(row0, _L)
        diff = (alpha + bu_v[chunk] + bi_v[chunk] + dot
                + a_v[chunk] * agec - r_v[chunk])
        return acc + diff * diff

    acc = lax.fori_loop(0, _NG, group, jnp.zeros((_L,), jnp.float32))
    accbuf[...] = acc
    pltpu.sync_copy(accbuf, out_hbm.at[wid])


@jax.jit
def _lfm_sc(sampleU, sampleI, sampleR, sampleA, al16, ag16,
            betaU, betaI, gammaU, gammaI):
    return pl.kernel(
        _lfm_body,
        out_type=jax.ShapeDtypeStruct((_NW, _L), jnp.float32),
        mesh=plsc.VectorSubcoreMesh(core_axis_name="c", subcore_axis_name="s"),
        compiler_params=pltpu.CompilerParams(
            needs_layout_passes=False, use_tc_tiling_on_sc=False),
        scratch_types=[
            pltpu.VMEM((_BPW,), jnp.int32),
            pltpu.VMEM((_BPW,), jnp.int32),
            pltpu.VMEM((_BPW,), jnp.float32),
            pltpu.VMEM((_BPW,), jnp.float32),
            pltpu.VMEM((_L,), jnp.float32),
            pltpu.VMEM((_L,), jnp.float32),
            pltpu.VMEM((_BPW,), jnp.float32),
            pltpu.VMEM((_BPW,), jnp.float32),
            pltpu.VMEM((_BPW, _K), jnp.float32),
            pltpu.VMEM((_BPW, _K), jnp.float32),
            pltpu.VMEM((_L * _L,), jnp.float32),
            pltpu.VMEM((_L,), jnp.float32),
            pltpu.SemaphoreType.DMA,
        ],
    )(sampleU, sampleI, sampleR, sampleA, al16, ag16,
      betaU, betaI, gammaU, gammaI)


def kernel(sampleU, sampleI, sampleR, sampleA, alpha, agecoef,
           betaU, betaI, gammaU, gammaI):
    al16 = jnp.full((_L,), alpha, jnp.float32)
    ag16 = jnp.full((_L,), agecoef, jnp.float32)
    partials = _lfm_sc(sampleU, sampleI, sampleR, sampleA, al16, ag16,
                       betaU, betaI, gammaU, gammaI)
    return jnp.sum(partials) * (1.0 / _B)


# speedup vs baseline: 4.3596x; 4.3596x over previous
import jax
import jax.numpy as jnp
from jax import lax
from jax.experimental import pallas as pl
from jax.experimental.pallas import tpu as pltpu
from jax.experimental.pallas import tpu_sc as plsc

_NC = 2
_NS = 16
_L = 16
_NW = _NC * _NS
_B = 16384
_K = 32
_BPW = _B // _NW
_NG = _BPW // _L
_RING = 8


def _lfm_body(u_hbm, i_hbm, r_hbm, a_hbm, al_hbm, ag_hbm,
              bu_hbm, bi_hbm, gut_hbm, git_hbm, out_hbm,
              idxu_v, idxi_v, r_v, a_v, al_v, ag_v,
              bu_v, bi_v, tu_ring, ti_ring,
              gu_a, gu_b, gi_a, gi_b, tbuf, accbuf, sem, gsem):
    wid = lax.axis_index("s") * _NC + lax.axis_index("c")
    base = wid * _BPW

    pltpu.sync_copy(u_hbm.at[pl.ds(base, _BPW)], idxu_v)
    pltpu.sync_copy(i_hbm.at[pl.ds(base, _BPW)], idxi_v)
    pltpu.sync_copy(r_hbm.at[pl.ds(base, _BPW)], r_v)
    pltpu.sync_copy(a_hbm.at[pl.ds(base, _BPW)], a_v)
    pltpu.sync_copy(al_hbm, al_v)
    pltpu.sync_copy(ag_hbm, ag_v)

    bcopies = [
        pltpu.async_copy(bu_hbm.at[idxu_v], bu_v, sem),
        pltpu.async_copy(bi_hbm.at[idxi_v], bi_v, sem),
    ]

    kio_a = lax.iota(jnp.int32, _L)
    kio_b = kio_a + _L

    def scalar_at(ref, j):
        vec = ref[pl.ds((j >> 4) * _L, _L)]
        return jnp.sum(jnp.where(kio_a == (j & 15), vec, 0))

    def fire(j, slot):
        cu = pl.multiple_of((scalar_at(idxu_v, j) >> 7) * 128, 128)
        ci = pl.multiple_of((scalar_at(idxi_v, j) >> 7) * 128, 128)
        pltpu.make_async_copy(
            gut_hbm.at[:, pl.ds(cu, 128)], tu_ring.at[slot], gsem).start()
        pltpu.make_async_copy(
            git_hbm.at[:, pl.ds(ci, 128)], ti_ring.at[slot], gsem).start()

    def wait_pair(slot):
        pltpu.make_async_copy(
            gut_hbm.at[:, pl.ds(0, 128)], tu_ring.at[slot], gsem).wait()
        pltpu.make_async_copy(
            git_hbm.at[:, pl.ds(0, 128)], ti_ring.at[slot], gsem).wait()

    for j in range(_RING):
        fire(j, j)

    def extract(j, _):
        slot = lax.rem(j, _RING)
        wait_pair(slot)
        lu = jnp.full((_L,), scalar_at(idxu_v, j) & 127, jnp.int32)
        li = jnp.full((_L,), scalar_at(idxi_v, j) & 127, jnp.int32)
        o = pl.ds(j * _L, _L)
        gu_a[o] = plsc.load_gather(tu_ring.at[slot], [kio_a, lu])
        gu_b[o] = plsc.load_gather(tu_ring.at[slot], [kio_b, lu])
        gi_a[o] = plsc.load_gather(ti_ring.at[slot], [kio_a, li])
        gi_b[o] = plsc.load_gather(ti_ring.at[slot], [kio_b, li])

        @pl.when(j < _BPW - _RING)
        def _():
            fire(j + _RING, slot)

        return 0

    lax.fori_loop(0, _BPW, extract, 0)
    for c in bcopies:
        c.wait()

    alpha = al_v[...]
    agec = ag_v[...]
    scat_base = lax.iota(jnp.int32, _L) * _L

    def group(g, acc):
        row0 = g * _L
        for s in range(_L):
            row = row0 + s
            rs = pl.ds(row * _L, _L)
            w = gu_a[rs] * gi_a[rs] + gu_b[rs] * gi_b[rs]
            plsc.store_scatter(tbuf, [scat_base + s], w)
        dot = tbuf[pl.ds(0, _L)]
        for d in range(1, _L):
            dot = dot + tbuf[pl.ds(d * _L, _L)]
        chunk = pl.ds(row0, _L)
        diff = (alpha + bu_v[chunk] + bi_v[chunk] + dot
                + a_v[chunk] * agec - r_v[chunk])
        return acc + diff * diff

    acc = lax.fori_loop(0, _NG, group, jnp.zeros((_L,), jnp.float32))
    accbuf[...] = acc
    pltpu.sync_copy(accbuf, out_hbm.at[wid])


@jax.jit
def _lfm_sc(sampleU, sampleI, sampleR, sampleA, al16, ag16,
            betaU, betaI, gammaU_t, gammaI_t):
    return pl.kernel(
        _lfm_body,
        out_type=jax.ShapeDtypeStruct((_NW, _L), jnp.float32),
        mesh=plsc.VectorSubcoreMesh(core_axis_name="c", subcore_axis_name="s"),
        compiler_params=pltpu.CompilerParams(
            needs_layout_passes=False, use_tc_tiling_on_sc=True),
        scratch_types=[
            pltpu.VMEM((_BPW,), jnp.int32),
            pltpu.VMEM((_BPW,), jnp.int32),
            pltpu.VMEM((_BPW,), jnp.float32),
            pltpu.VMEM((_BPW,), jnp.float32),
            pltpu.VMEM((_L,), jnp.float32),
            pltpu.VMEM((_L,), jnp.float32),
            pltpu.VMEM((_BPW,), jnp.float32),
            pltpu.VMEM((_BPW,), jnp.float32),
            pltpu.VMEM((_RING, _K, 128), jnp.float32),
            pltpu.VMEM((_RING, _K, 128), jnp.float32),
            pltpu.VMEM((_BPW * _L,), jnp.float32),
            pltpu.VMEM((_BPW * _L,), jnp.float32),
            pltpu.VMEM((_BPW * _L,), jnp.float32),
            pltpu.VMEM((_BPW * _L,), jnp.float32),
            pltpu.VMEM((_L * _L,), jnp.float32),
            pltpu.VMEM((_L,), jnp.float32),
            pltpu.SemaphoreType.DMA,
            pltpu.SemaphoreType.DMA,
        ],
    )(sampleU, sampleI, sampleR, sampleA, al16, ag16,
      betaU, betaI, gammaU_t, gammaI_t)


def kernel(sampleU, sampleI, sampleR, sampleA, alpha, agecoef,
           betaU, betaI, gammaU, gammaI):
    al16 = jnp.full((_L,), alpha, jnp.float32)
    ag16 = jnp.full((_L,), agecoef, jnp.float32)
    partials = _lfm_sc(sampleU, sampleI, sampleR, sampleA, al16, ag16,
                       betaU, betaI, gammaU.T, gammaI.T)
    return jnp.sum(partials) * (1.0 / _B)
